# baseline (speedup 1.0000x reference)
import functools

import jax
import jax.numpy as jnp
from jax import lax
from jax.experimental import pallas as pl
from jax.experimental.pallas import tpu as pltpu
from jax.experimental.pallas import tpu_sc as plsc

VOCAB = 100000
EMB = 300
B = 1024
L = 200
HID = 256
TGT = 20

NBUF = 5
NW = 32
ROWS_PER_W = B // NW
SB_DESC = 25
SB_ROWS = 16
IMG_ROWS_W = 2 * SB_DESC
W2COLS = EMB - 256

_SPLITS = []
for _p in range(SB_DESC):
    _s = (128 * _p) % 200
    _k = (128 * _p) // 200
    if _s + 128 < 200:
        _SPLITS.append((None, _k))
    else:
        _SPLITS.append((200 - _s, _k))
_SPLITS = tuple(_SPLITS)

_FULL_CHUNKS = tuple((16 * k, k) for k in range(8))
_TAIL_CHUNKS = ((0, 0), (16, 1), (28, 2))


def _make_pool(chunks, nacc, ncols):
    def accumulate(buf, accs, r0, r1):
        def abody(r, accs):
            sums, maxs = accs
            sums, maxs = list(sums), list(maxs)
            for off, ai in chunks:
                v = buf[r, pl.ds(off, 16)]
                sums[ai] = sums[ai] + v
                maxs[ai] = jnp.maximum(maxs[ai], v)
            return (tuple(sums), tuple(maxs))

        return lax.fori_loop(r0, r1, abody, accs, unroll=2)

    def body(xim_hbm, t_hbm, out_hbm, idx_v, b0, b1, b2, b3, b4, stage,
             s0, s1, s2, s3, s4):
        bufs = (b0, b1, b2, b3, b4)
        sems = (s0, s1, s2, s3, s4)
        cid = lax.axis_index("c")
        sid = lax.axis_index("s")
        w = sid * 2 + cid

        def src(g):
            return t_hbm.at[idx_v.at[g]]

        def fresh():
            return (
                tuple(jnp.zeros((16,), jnp.float32) for _ in range(nacc)),
                tuple(jnp.full((16,), -jnp.inf, jnp.float32)
                      for _ in range(nacc)),
            )

        inv_l = jnp.float32(1.0 / L)

        def finalize(accs, row):
            sums, maxs = accs
            if ncols == 128:
                for i in range(8):
                    stage[pl.ds(16 * i, 16)] = sums[i] * inv_l
                    stage[pl.ds(128 + 16 * i, 16)] = maxs[i]
            else:
                stage[pl.ds(W2COLS - 16, 16)] = sums[2] * inv_l
                stage[pl.ds(2 * W2COLS - 16, 16)] = maxs[2]
                for i in range(2):
                    stage[pl.ds(16 * i, 16)] = sums[i] * inv_l
                    stage[pl.ds(W2COLS + 16 * i, 16)] = maxs[i]
            pltpu.sync_copy(stage, out_hbm.at[row])

        pltpu.sync_copy(xim_hbm.at[pl.ds(w * IMG_ROWS_W, IMG_ROWS_W)], idx_v)
        for p in range(NBUF):
            pltpu.async_copy(src(p), bufs[p], sems[p])

        def sb_body(sb, carry):
            g0 = sb * SB_DESC
            accs = fresh()
            for p in range(SB_DESC):
                bi = p % NBUF
                pltpu.make_async_copy(src(g0 + p), bufs[bi], sems[bi]).wait()
                m, k = _SPLITS[p]
                if m is None:
                    accs = accumulate(bufs[bi], accs, 0, 128)
                else:
                    accs = accumulate(bufs[bi], accs, 0, m)
                    finalize(accs, w * ROWS_PER_W + sb * SB_ROWS + k)
                    accs = fresh()
                    if m < 128:
                        accs = accumulate(bufs[bi], accs, m, 128)
                nxt = jnp.minimum(g0 + p + NBUF, 2 * SB_DESC - 1)
                pltpu.async_copy(src(nxt), bufs[bi], sems[bi])
            return carry

        lax.fori_loop(0, 2, sb_body, None)

        for p in range(NBUF):
            pltpu.make_async_copy(src(2 * SB_DESC - 1), bufs[p],
                                  sems[p]).wait()

    return functools.partial(
        pl.kernel,
        out_type=jax.ShapeDtypeStruct((B, 2 * ncols), jnp.float32),
        mesh=plsc.VectorSubcoreMesh(core_axis_name="c", subcore_axis_name="s"),
        compiler_params=pltpu.CompilerParams(use_tc_tiling_on_sc=False),
        scratch_types=(
            [pltpu.VMEM((IMG_ROWS_W, 128), jnp.int32)]
            + [pltpu.VMEM((128, 128), jnp.float32) for _ in range(NBUF)]
            + [pltpu.VMEM((2 * ncols,), jnp.float32)]
            + [pltpu.SemaphoreType.DMA for _ in range(NBUF)]
        ),
    )(body)


_pool_full = _make_pool(_FULL_CHUNKS, 8, 128)
_pool_tail = _make_pool(_TAIL_CHUNKS, 3, W2COLS)


def _mlp_body(h0_ref, h1_ref, h2_ref, g1_ref, b1_ref, w1t_ref, bias1_ref,
              g2_ref, b2_ref, w2t_ref, bias2_ref, out_ref, hid_ref):
    h0 = h0_ref[...]
    h1v = h1_ref[...]
    h2v = h2_ref[...]
    h = jnp.concatenate(
        [h0[:, :128], h1v[:, :128], h2v[:, :W2COLS],
         h0[:, 128:], h1v[:, 128:], h2v[:, W2COLS:]], axis=1)
    mu = jnp.mean(h, axis=0, keepdims=True)
    d = h - mu
    var = jnp.mean(d * d, axis=0, keepdims=True)
    hn = d * lax.rsqrt(var + 1e-5) * g1_ref[...] + b1_ref[...]
    h1 = jnp.dot(hn, w1t_ref[...], preferred_element_type=jnp.float32,
                 precision=lax.Precision.HIGHEST) + bias1_ref[...]
    hid_ref[...] = h1
    mu2 = jnp.mean(h1, axis=0, keepdims=True)
    d2 = h1 - mu2
    var2 = jnp.mean(d2 * d2, axis=0, keepdims=True)
    h2 = d2 * lax.rsqrt(var2 + 1e-5) * g2_ref[...] + b2_ref[...]
    out_ref[...] = jnp.dot(h2, w2t_ref[...], preferred_element_type=jnp.float32,
                           precision=lax.Precision.HIGHEST) + bias2_ref[...]


_mlp = pl.pallas_call(
    _mlp_body,
    out_shape=(
        jax.ShapeDtypeStruct((B, TGT), jnp.float32),
        jax.ShapeDtypeStruct((B, HID), jnp.float32),
    ),
)


def kernel(x, emb, g1, b1, W1, bias1, g2, b2, W2, bias2):
    x_im = x.reshape(B * L // 128, 128)
    t0 = emb[:, 0:128]
    t1 = emb[:, 128:256]
    t2 = jnp.pad(emb[:, 256:EMB], ((0, 0), (0, 128 - W2COLS)))
    h0 = _pool_full(x_im, t0)
    h1 = _pool_full(x_im, t1)
    h2 = _pool_tail(x_im, t2)
    out, hid = _mlp(h0, h1, h2, g1.reshape(1, -1), b1.reshape(1, -1), W1.T,
                    bias1.reshape(1, -1), g2.reshape(1, -1),
                    b2.reshape(1, -1), W2.T, bias2.reshape(1, -1))
    return (out, hid)


# speedup vs baseline: 1.0027x; 1.0027x over previous
import functools

import jax
import jax.numpy as jnp
from jax import lax
from jax.experimental import pallas as pl
from jax.experimental.pallas import tpu as pltpu
from jax.experimental.pallas import tpu_sc as plsc

VOCAB = 100000
EMB = 300
B = 1024
L = 200
HID = 256
TGT = 20

NCHUNK = 5
CHUNK = L // NCHUNK
NW = 32
ROWS_PER_W = B // NW
IDX_ROWS = ROWS_PER_W * NCHUNK
W2COLS = EMB - 256

_FULL_CHUNKS = tuple((16 * k, k) for k in range(8))
_TAIL_CHUNKS = ((0, 0), (16, 1), (28, 2))


def _make_pool(chunks, nacc, ncols):
    def accumulate(buf, accs):
        def abody(r, accs):
            sums, maxs = accs
            sums, maxs = list(sums), list(maxs)
            for off, ai in chunks:
                v = buf[r, pl.ds(off, 16)]
                sums[ai] = sums[ai] + v
                maxs[ai] = jnp.maximum(maxs[ai], v)
            return (tuple(sums), tuple(maxs))

        return lax.fori_loop(0, CHUNK, abody, accs, unroll=2)

    def body(x2_hbm, t_hbm, out_hbm, idx_v, b0, b1, b2, b3, b4, stage,
             s0, s1, s2, s3, s4):
        bufs = (b0, b1, b2, b3, b4)
        sems = (s0, s1, s2, s3, s4)
        cid = lax.axis_index("c")
        sid = lax.axis_index("s")
        w = sid * 2 + cid

        def src(row):
            return t_hbm.at[idx_v.at[row]]

        pltpu.sync_copy(x2_hbm.at[pl.ds(w * IDX_ROWS, IDX_ROWS)], idx_v)
        for j in range(NCHUNK):
            pltpu.async_copy(src(j), bufs[j], sems[j])

        inv_l = jnp.float32(1.0 / L)

        def row_body(b, carry):
            accs = (
                tuple(jnp.zeros((16,), jnp.float32) for _ in range(nacc)),
                tuple(jnp.full((16,), -jnp.inf, jnp.float32)
                      for _ in range(nacc)),
            )
            for j in range(NCHUNK):
                pltpu.make_async_copy(src(b * NCHUNK + j), bufs[j],
                                      sems[j]).wait()
                accs = accumulate(bufs[j], accs)
                nxt = jnp.minimum(b + 1, ROWS_PER_W - 1) * NCHUNK + j
                pltpu.async_copy(src(nxt), bufs[j], sems[j])

            sums, maxs = accs
            if ncols == 128:
                for i in range(8):
                    stage[pl.ds(16 * i, 16)] = sums[i] * inv_l
                    stage[pl.ds(128 + 16 * i, 16)] = maxs[i]
            else:
                stage[pl.ds(W2COLS - 16, 16)] = sums[2] * inv_l
                stage[pl.ds(2 * W2COLS - 16, 16)] = maxs[2]
                for i in range(2):
                    stage[pl.ds(16 * i, 16)] = sums[i] * inv_l
                    stage[pl.ds(W2COLS + 16 * i, 16)] = maxs[i]
            pltpu.sync_copy(stage, out_hbm.at[w * ROWS_PER_W + b])
            return carry

        lax.fori_loop(0, ROWS_PER_W, row_body, None)

        for j in range(NCHUNK):
            pltpu.make_async_copy(src((ROWS_PER_W - 1) * NCHUNK + j),
                                  bufs[j], sems[j]).wait()

    return functools.partial(
        pl.kernel,
        out_type=jax.ShapeDtypeStruct((B, 2 * ncols), jnp.float32),
        mesh=plsc.VectorSubcoreMesh(core_axis_name="c", subcore_axis_name="s"),
        compiler_params=pltpu.CompilerParams(use_tc_tiling_on_sc=False),
        scratch_types=(
            [pltpu.VMEM((IDX_ROWS, CHUNK), jnp.int32)]
            + [pltpu.VMEM((CHUNK, 128), jnp.float32) for _ in range(NCHUNK)]
            + [pltpu.VMEM((2 * ncols,), jnp.float32)]
            + [pltpu.SemaphoreType.DMA for _ in range(NCHUNK)]
        ),
    )(body)


_pool_full = _make_pool(_FULL_CHUNKS, 8, 128)
_pool_tail = _make_pool(_TAIL_CHUNKS, 3, W2COLS)


def _mlp_body(h0_ref, h1_ref, h2_ref, g1_ref, b1_ref, w1t_ref, bias1_ref,
              g2_ref, b2_ref, w2t_ref, bias2_ref, out_ref, hid_ref):
    h0 = h0_ref[...]
    h1v = h1_ref[...]
    h2v = h2_ref[...]
    h = jnp.concatenate(
        [h0[:, :128], h1v[:, :128], h2v[:, :W2COLS],
         h0[:, 128:], h1v[:, 128:], h2v[:, W2COLS:]], axis=1)
    mu = jnp.mean(h, axis=0, keepdims=True)
    d = h - mu
    var = jnp.mean(d * d, axis=0, keepdims=True)
    hn = d * lax.rsqrt(var + 1e-5) * g1_ref[...] + b1_ref[...]
    h1 = jnp.dot(hn, w1t_ref[...], preferred_element_type=jnp.float32,
                 precision=lax.Precision.HIGHEST) + bias1_ref[...]
    hid_ref[...] = h1
    mu2 = jnp.mean(h1, axis=0, keepdims=True)
    d2 = h1 - mu2
    var2 = jnp.mean(d2 * d2, axis=0, keepdims=True)
    h2 = d2 * lax.rsqrt(var2 + 1e-5) * g2_ref[...] + b2_ref[...]
    out_ref[...] = jnp.dot(h2, w2t_ref[...], preferred_element_type=jnp.float32,
                           precision=lax.Precision.HIGHEST) + bias2_ref[...]


_mlp = pl.pallas_call(
    _mlp_body,
    out_shape=(
        jax.ShapeDtypeStruct((B, TGT), jnp.float32),
        jax.ShapeDtypeStruct((B, HID), jnp.float32),
    ),
)


def kernel(x, emb, g1, b1, W1, bias1, g2, b2, W2, bias2):
    x2 = x.reshape(B * NCHUNK, CHUNK)
    t0 = emb[:, 0:128]
    t1 = emb[:, 128:256]
    t2 = jnp.pad(emb[:, 256:EMB], ((0, 0), (0, 128 - W2COLS)))
    h0 = _pool_full(x2, t0)
    h1 = _pool_full(x2, t1)
    h2 = _pool_tail(x2, t2)
    out, hid = _mlp(h0, h1, h2, g1.reshape(1, -1), b1.reshape(1, -1), W1.T,
                    bias1.reshape(1, -1), g2.reshape(1, -1),
                    b2.reshape(1, -1), W2.T, bias2.reshape(1, -1))
    return (out, hid)
